# Initial kernel scaffold; baseline (speedup 1.0000x reference)
import jax
import jax.numpy as jnp
from jax.experimental import pallas as pl


def kernel(min_embedding, delta_embedding, relation_embedding, scaling_embedding, data0, data1, data2, data3, data4, data5, data6):
    raise NotImplementedError("write your pallas kernel here")


# speedup vs baseline: 1.3557x; 1.3557x over previous
import functools
import math

import jax
import jax.numpy as jnp
from jax.experimental import pallas as pl
from jax.experimental.pallas import tpu as pltpu

_EPS = 1e-08
_LOG_LO = math.log(1e-10)
_LOG_HI = math.log(1e4)
_DIM = 64
_NB = 1024
_INTERPRET = False


def _lv(w):
    sp = jnp.maximum(w, 0.0) + jnp.log1p(jnp.exp(-jnp.abs(w)))
    s = jnp.sum(jnp.log(sp), axis=-1)
    return jnp.clip(s, _LOG_LO, _LOG_HI)


def _reg_sums(mn, mx):
    s1 = jnp.sum(jnp.maximum(mx - 1.0 + _EPS, 0.0))
    s2 = jnp.sum(mn * mn)
    return s1, s2


def _compute_body(gmin_ref, gdel_ref, grel_ref, gscal_ref, out_ref):
    step = pl.program_id(0)
    mn = gmin_ref[...]
    mx = mn + jnp.exp(gdel_ref[...])
    rel = grel_ref[...]
    sc = gscal_ref[...] + _EPS

    parts = []

    def box_pair(a, b):
        return (mn[a], mx[a]), (mn[b], mx[b])

    (mn1, mx1), (mn2, mx2) = box_pair(0, 1)
    w_i = jnp.minimum(mx1, mx2) - jnp.maximum(mn1, mn2)
    nf1 = jnp.sum(1.0 - jnp.exp(_lv(w_i) - _lv(mx1 - mn1)))
    r_nf1 = [_reg_sums(mn1, mx1), _reg_sums(mn2, mx2)]

    (mn1, mx1), (mn2, mx2) = box_pair(2, 3)
    mn3, mx3 = mn[4], mx[4]
    mni = jnp.maximum(mn1, mn2)
    mxi = jnp.minimum(mx1, mx2)
    w_i123 = jnp.minimum(mxi, mx3) - jnp.maximum(mni, mn3)
    nf2 = jnp.sum(1.0 - jnp.exp(_lv(w_i123) - _lv(mxi - mni)))
    r_nf2 = [_reg_sums(mni, mxi), _reg_sums(mn1, mx1),
             _reg_sums(mn2, mx2), _reg_sums(mn3, mx3)]

    (mn1, mx1), (mn2, mx2) = box_pair(5, 6)
    tmn = mn1 * sc[0] + rel[0]
    tmx = mx1 * sc[0] + rel[0]
    w_i = jnp.minimum(tmx, mx2) - jnp.maximum(tmn, mn2)
    nf3 = jnp.sum(1.0 - jnp.exp(_lv(w_i) - _lv(tmx - tmn)))
    r_nf3 = [_reg_sums(tmn, tmx), _reg_sums(mn1, mx1), _reg_sums(mn2, mx2)]

    (mn1, mx1), (mn2, mx2) = box_pair(7, 8)
    tmn = (mn1 - rel[1]) / sc[1]
    tmx = (mx1 - rel[1]) / sc[1]
    w_i = jnp.minimum(tmx, mx2) - jnp.maximum(tmn, mn2)
    nf4 = jnp.sum(1.0 - jnp.exp(_lv(w_i) - _lv(tmx - tmn)))
    r_nf4 = [_reg_sums(tmn, tmx), _reg_sums(mn1, mx1), _reg_sums(mn2, mx2)]

    (mn1, mx1), (mn2, mx2) = box_pair(9, 10)
    w_i = jnp.minimum(mx1, mx2) - jnp.maximum(mn1, mn2)
    dj = jnp.sum(jnp.exp(_lv(w_i) - (_lv(mx1 - mn1) + _lv(mx2 - mn2))))
    r_dj = [_reg_sums(mn1, mx1), _reg_sums(mn2, mx2)]

    (mn1, mx1), (mn2, mx2) = box_pair(11, 12)
    tmn = mn1 * sc[2] + rel[2]
    tmx = mx1 * sc[2] + rel[2]
    w_i = jnp.minimum(tmx, mx2) - jnp.maximum(tmn, mn2)
    nf3n = jnp.sum(jnp.exp(_lv(w_i) - _lv(tmx - tmn)))
    r_nf3n = [_reg_sums(tmn, tmx), _reg_sums(mn1, mx1), _reg_sums(mn2, mx2)]

    parts = [nf1, nf2, nf3, nf4, dj, nf3n]
    for s1, s2 in (r_nf1 + r_nf2 + r_nf3 + r_nf4 + r_dj + r_nf3n):
        parts.append(s1)
        parts.append(s2)
    vals = jnp.concatenate([jnp.stack(parts),
                            jnp.zeros((128 - len(parts),), jnp.float32)])

    @pl.when(step == 0)
    def _():
        out_ref[...] = vals

    @pl.when(step != 0)
    def _():
        out_ref[...] += vals


def _compute_partials(gmin, gdel, grel, gscal):
    b = gmin.shape[1]
    grid = (b // _NB,)
    return pl.pallas_call(
        _compute_body,
        grid=grid,
        in_specs=[
            pl.BlockSpec((13, _NB, _DIM), lambda i: (0, i, 0)),
            pl.BlockSpec((13, _NB, _DIM), lambda i: (0, i, 0)),
            pl.BlockSpec((3, _NB, _DIM), lambda i: (0, i, 0)),
            pl.BlockSpec((3, _NB, _DIM), lambda i: (0, i, 0)),
        ],
        out_specs=pl.BlockSpec((128,), lambda i: (0,)),
        out_shape=jax.ShapeDtypeStruct((128,), jnp.float32),
        interpret=_INTERPRET,
    )(gmin, gdel, grel, gscal)


def kernel(min_embedding, delta_embedding, relation_embedding, scaling_embedding,
           data0, data1, data2, data3, data4, data5, data6):
    b = data0.shape[0]
    idx_all = jnp.concatenate([
        data0[:, 0], data0[:, 2],
        data1[:, 0], data1[:, 1], data1[:, 2],
        data2[:, 0], data2[:, 2],
        data3[:, 1], data3[:, 2],
        data4[:, 0], data4[:, 1],
        data6[:, 0], data6[:, 2],
    ])
    idx_rel = jnp.concatenate([data2[:, 1], data3[:, 0], data6[:, 1]])

    gmin = jnp.take(min_embedding, idx_all, axis=0).reshape(13, b, _DIM)
    gdel = jnp.take(delta_embedding, idx_all, axis=0).reshape(13, b, _DIM)
    grel = jnp.take(relation_embedding, idx_rel, axis=0).reshape(3, b, _DIM)
    gscal = jnp.take(scaling_embedding, idx_rel, axis=0).reshape(3, b, _DIM)

    s = _compute_partials(gmin, gdel, grel, gscal)

    denom = float(b * _DIM)

    def l2s(j):
        s1 = s[6 + 2 * j]
        s2 = s[7 + 2 * j]
        return s1 / denom + jnp.maximum(jnp.sqrt(s2) - 1.0, 0.0)

    nf1_reg = l2s(0) + l2s(1)
    nf2_reg = l2s(2) + l2s(3) + l2s(4) + l2s(5)
    nf3_reg = l2s(6) + l2s(7) + l2s(8)
    nf4_reg = l2s(9) + l2s(10) + l2s(11)
    dj_reg = l2s(12) + l2s(13)
    nf3n_reg = l2s(14) + l2s(15) + l2s(16)
    return (s[0], s[1], s[2], s[3], s[4], s[5],
            nf1_reg, nf2_reg, nf3_reg, nf4_reg, dj_reg, nf3n_reg)
